# baseline (speedup 1.0000x reference)
import functools

import jax
import jax.numpy as jnp
import numpy as np
from jax.experimental import pallas as pl
from jax.experimental.pallas import tpu as pltpu
from jax.sharding import Mesh, PartitionSpec as P


def _phase1_kernel(x_ref, w1_ref, b1_ref, a_ref, g_ref, cmat_ref, *,
                   n_tile, n_valid):
    t = pl.program_id(1)

    x = x_ref[0].astype(jnp.bfloat16)
    w1 = w1_ref[...].astype(jnp.bfloat16)
    g = jnp.dot(w1, x, preferred_element_type=jnp.float32) + b1_ref[...]
    alpha = a_ref[0]
    g = jnp.where(g >= 0.0, g, alpha * g)

    if n_valid is not None:
        col = jax.lax.broadcasted_iota(jnp.int32, g.shape, 1) + t * n_tile
        g = jnp.where(col < n_valid, g, 0.0)

    gb = g.astype(jnp.bfloat16)
    g_ref[0] = gb

    @pl.when(t == 0)
    def _():
        cmat_ref[0] = jnp.zeros_like(cmat_ref[0])

    gram = jax.lax.dot_general(gb, gb, (((1,), (1,)), ((), ())),
                               preferred_element_type=jnp.float32)
    cmat_ref[0] = cmat_ref[0] + gram


def _phase2_kernel(x_ref, g_ref, wproj_ref, w2_ref, b2_ref, o_ref):
    x = x_ref[0]
    gb = g_ref[0]
    wp = wproj_ref[0].astype(jnp.bfloat16)
    y = jax.nn.sigmoid(
        jnp.dot(wp, gb, preferred_element_type=jnp.float32))
    w2 = w2_ref[...].astype(jnp.bfloat16)
    att = jnp.dot(w2, y.astype(jnp.bfloat16),
                  preferred_element_type=jnp.float32) + b2_ref[...]
    o_ref[0] = (x * jnp.tanh(att)).astype(o_ref.dtype)


def _sapca_impl(x, w1, b1, a, w2, b2, *, batch_scale):
    B, C, Hh, Ww = x.shape
    N = Hh * Ww
    Ci = w2.shape[1]

    N_pad = ((N + 127) // 128) * 128
    n_tile = N_pad
    max_tile = max(128, ((4 << 20) // (4 * C)) // 128 * 128)
    while n_tile > max_tile and n_tile % 2 == 0 and (n_tile // 2) % 128 == 0:
        n_tile //= 2
    nt = N_pad // n_tile
    n_valid = N if N_pad != N else None

    xf = x.reshape(B, C, N)
    if N_pad != N:
        xf = jnp.pad(xf, ((0, 0), (0, 0), (0, N_pad - N)))

    const = lambda b, t: (0, 0)
    smem_spec = pl.BlockSpec(memory_space=pltpu.MemorySpace.SMEM)
    vmem_limit = 64 * 1024 * 1024

    g_saved, cmat = pl.pallas_call(
        functools.partial(_phase1_kernel, n_tile=n_tile, n_valid=n_valid),
        out_shape=(
            jax.ShapeDtypeStruct((B, C, N_pad), jnp.bfloat16),
            jax.ShapeDtypeStruct((B, C, C), jnp.float32),
        ),
        grid_spec=pltpu.PrefetchScalarGridSpec(
            num_scalar_prefetch=0,
            grid=(B, nt),
            in_specs=[
                pl.BlockSpec((1, C, n_tile), lambda b, t: (b, 0, t)),
                pl.BlockSpec((C, C), const),
                pl.BlockSpec((C, 1), const),
                smem_spec,
            ],
            out_specs=(
                pl.BlockSpec((1, C, n_tile), lambda b, t: (b, 0, t)),
                pl.BlockSpec((1, C, C), lambda b, t: (b, 0, 0)),
            ),
        ),
        compiler_params=pltpu.CompilerParams(
            dimension_semantics=("parallel", "arbitrary"),
            vmem_limit_bytes=vmem_limit),
    )(xf, w1, b1, a)

    _, evecs = jnp.linalg.eigh(cmat * (1.0 / batch_scale))
    wproj = jnp.swapaxes(evecs[:, :, C - Ci:], -1, -2)

    out = pl.pallas_call(
        _phase2_kernel,
        out_shape=jax.ShapeDtypeStruct((B, C, N_pad), jnp.float32),
        grid_spec=pltpu.PrefetchScalarGridSpec(
            num_scalar_prefetch=0,
            grid=(B, nt),
            in_specs=[
                pl.BlockSpec((1, C, n_tile), lambda b, t: (b, 0, t)),
                pl.BlockSpec((1, C, n_tile), lambda b, t: (b, 0, t)),
                pl.BlockSpec((1, Ci, C), lambda b, t: (b, 0, 0)),
                pl.BlockSpec((C, Ci), const),
                pl.BlockSpec((C, 1), const),
            ],
            out_specs=pl.BlockSpec((1, C, n_tile), lambda b, t: (b, 0, t)),
        ),
        compiler_params=pltpu.CompilerParams(
            dimension_semantics=("parallel", "parallel"),
            vmem_limit_bytes=vmem_limit),
    )(xf, g_saved, wproj, w2, b2)

    if N_pad != N:
        out = out[:, :, :N]
    return out.reshape(B, C, Hh, Ww)


def kernel(x, w1, b1, a, w2, b2):
    B = x.shape[0]
    devs = jax.devices()
    n_dev = min(2, len(devs))
    if n_dev == 2 and B % 2 == 0:
        mesh = Mesh(np.array([devs[1], devs[0]]), ("b",))
        fn = jax.shard_map(
            functools.partial(_sapca_impl, batch_scale=B),
            mesh=mesh,
            in_specs=(P("b"), P(), P(), P(), P(), P()),
            out_specs=P("b"),
            check_vma=False,
        )
        return fn(x, w1, b1, a, w2, b2)
    return _sapca_impl(x, w1, b1, a, w2, b2, batch_scale=B)


# speedup vs baseline: 1.0147x; 1.0147x over previous
import functools

import jax
import jax.numpy as jnp
import numpy as np
from jax.core import ShapedArray
from jax.experimental import pallas as pl
from jax.experimental.pallas import tpu as pltpu
from jax.sharding import Mesh, PartitionSpec as P
import jax.extend as jex
from jax._src.interpreters import mlir as _mlir

_eigh_cc_p = jex.core.Primitive("sapca_eigh_cc")
_eigh_cc_p.multiple_results = True


def _eigh_cc_abstract(a):
    b, n = a.shape[:-2], a.shape[-1]
    return (ShapedArray(a.shape, a.dtype), ShapedArray(b + (n,), a.dtype))


_eigh_cc_p.def_abstract_eval(_eigh_cc_abstract)


def _eigh_cc_lowering(ctx, operand):
    v_aval, w_aval = ctx.avals_out
    result_types = [_mlir.aval_to_ir_type(ctx.module_context, v_aval),
                    _mlir.aval_to_ir_type(ctx.module_context, w_aval)]
    op = _mlir.custom_call("Eigh", result_types=result_types,
                           operands=[operand],
                           backend_config="1,1,100,1e-6",
                           api_version=1)
    return op.results


_mlir.register_lowering(_eigh_cc_p, _eigh_cc_lowering, platform="tpu")


def _eigh_vectors_sorted(a):
    v, _ = _eigh_cc_p.bind(a)
    return v


def _phase1_kernel(x_ref, w1_ref, b1_ref, a_ref, g_ref, cmat_ref, *,
                   n_tile, n_valid, nt, inv_b):
    t = pl.program_id(1)

    x = x_ref[0].astype(jnp.bfloat16)
    w1 = w1_ref[...].astype(jnp.bfloat16)
    g = jnp.dot(w1, x, preferred_element_type=jnp.float32) + b1_ref[...]
    alpha = a_ref[0]
    g = jnp.where(g >= 0.0, g, alpha * g)

    if n_valid is not None:
        col = jax.lax.broadcasted_iota(jnp.int32, g.shape, 1) + t * n_tile
        g = jnp.where(col < n_valid, g, 0.0)

    gb = g.astype(jnp.bfloat16)
    g_ref[0] = gb

    @pl.when(t == 0)
    def _():
        cmat_ref[0] = jnp.zeros_like(cmat_ref[0])

    gram = jax.lax.dot_general(gb, gb, (((1,), (1,)), ((), ())),
                               preferred_element_type=jnp.float32)
    cmat_ref[0] = cmat_ref[0] + gram

    @pl.when(t == nt - 1)
    def _():
        acc = cmat_ref[0] * inv_b
        row = jax.lax.broadcasted_iota(jnp.int32, acc.shape, 0)
        colj = jax.lax.broadcasted_iota(jnp.int32, acc.shape, 1)
        cmat_ref[0] = jnp.where(row >= colj, acc, acc.T)


def _phase2_kernel(x_ref, g_ref, wproj_ref, w2_ref, b2_ref, o_ref):
    x = x_ref[0]
    gb = g_ref[0]
    wp = wproj_ref[0].astype(jnp.bfloat16)
    y = jax.nn.sigmoid(
        jnp.dot(wp, gb, preferred_element_type=jnp.float32))
    w2 = w2_ref[...].astype(jnp.bfloat16)
    att = jnp.dot(w2, y.astype(jnp.bfloat16),
                  preferred_element_type=jnp.float32) + b2_ref[...]
    o_ref[0] = (x * jnp.tanh(att)).astype(o_ref.dtype)


def _sapca_impl(x, w1, b1, a, w2, b2, *, batch_scale):
    B, C, Hh, Ww = x.shape
    N = Hh * Ww
    Ci = w2.shape[1]

    N_pad = ((N + 127) // 128) * 128
    n_tile = N_pad
    max_tile = max(128, ((4 << 20) // (4 * C)) // 128 * 128)
    while n_tile > max_tile and n_tile % 2 == 0 and (n_tile // 2) % 128 == 0:
        n_tile //= 2
    nt = N_pad // n_tile
    n_valid = N if N_pad != N else None

    xf = x.reshape(B, C, N)
    if N_pad != N:
        xf = jnp.pad(xf, ((0, 0), (0, 0), (0, N_pad - N)))

    const = lambda b, t: (0, 0)
    smem_spec = pl.BlockSpec(memory_space=pltpu.MemorySpace.SMEM)
    vmem_limit = 64 * 1024 * 1024

    g_saved, cmat = pl.pallas_call(
        functools.partial(_phase1_kernel, n_tile=n_tile, n_valid=n_valid,
                          nt=nt, inv_b=1.0 / batch_scale),
        out_shape=(
            jax.ShapeDtypeStruct((B, C, N_pad), jnp.bfloat16),
            jax.ShapeDtypeStruct((B, C, C), jnp.float32),
        ),
        grid_spec=pltpu.PrefetchScalarGridSpec(
            num_scalar_prefetch=0,
            grid=(B, nt),
            in_specs=[
                pl.BlockSpec((1, C, n_tile), lambda b, t: (b, 0, t)),
                pl.BlockSpec((C, C), const),
                pl.BlockSpec((C, 1), const),
                smem_spec,
            ],
            out_specs=(
                pl.BlockSpec((1, C, n_tile), lambda b, t: (b, 0, t)),
                pl.BlockSpec((1, C, C), lambda b, t: (b, 0, 0)),
            ),
        ),
        compiler_params=pltpu.CompilerParams(
            dimension_semantics=("parallel", "arbitrary"),
            vmem_limit_bytes=vmem_limit),
    )(xf, w1, b1, a)

    evecs = _eigh_vectors_sorted(cmat)
    wproj = jnp.swapaxes(evecs[:, :, C - Ci:], -1, -2)

    out = pl.pallas_call(
        _phase2_kernel,
        out_shape=jax.ShapeDtypeStruct((B, C, N_pad), jnp.float32),
        grid_spec=pltpu.PrefetchScalarGridSpec(
            num_scalar_prefetch=0,
            grid=(B, nt),
            in_specs=[
                pl.BlockSpec((1, C, n_tile), lambda b, t: (b, 0, t)),
                pl.BlockSpec((1, C, n_tile), lambda b, t: (b, 0, t)),
                pl.BlockSpec((1, Ci, C), lambda b, t: (b, 0, 0)),
                pl.BlockSpec((C, Ci), const),
                pl.BlockSpec((C, 1), const),
            ],
            out_specs=pl.BlockSpec((1, C, n_tile), lambda b, t: (b, 0, t)),
        ),
        compiler_params=pltpu.CompilerParams(
            dimension_semantics=("parallel", "parallel"),
            vmem_limit_bytes=vmem_limit),
    )(xf, g_saved, wproj, w2, b2)

    if N_pad != N:
        out = out[:, :, :N]
    return out.reshape(B, C, Hh, Ww)


def kernel(x, w1, b1, a, w2, b2):
    B = x.shape[0]
    devs = jax.devices()
    if len(devs) >= 2 and B % 2 == 0:
        mesh = Mesh(np.array(devs[:2]), ("b",))
        fn = jax.shard_map(
            functools.partial(_sapca_impl, batch_scale=B),
            mesh=mesh,
            in_specs=(P("b"), P(), P(), P(), P(), P()),
            out_specs=P("b"),
            check_vma=False,
        )
        return fn(x, w1, b1, a, w2, b2)
    return _sapca_impl(x, w1, b1, a, w2, b2, batch_scale=B)
